# baseline (speedup 1.0000x reference)
import functools

import jax
import jax.numpy as jnp
from jax import lax
from jax.experimental import pallas as pl
from jax.experimental.pallas import tpu as pltpu
from jax.experimental.pallas import tpu_sc as plsc

E = 8
K = 2
D = 2048
F = 8192
T = 4096
C = 1024
S = E * C

TB = 512
NB = T // TB
EP = 128

FB = 512
NF = F // FB

NC, NS, L = 2, 16, 16
NW = NC * NS

_SC_MESH = dict(core_axis_name="c", subcore_axis_name="s",
                num_cores=NC, num_subcores=NS)


def _route_body(x_ref, rk_ref, slot_ref, gate_ref, carry_ref):
    g = pl.program_id(0)
    k = g // NB

    @pl.when(g == 0)
    def _():
        carry_ref[...] = jnp.zeros_like(carry_ref)

    xb = x_ref[...]
    rk = rk_ref[...]
    logits = jnp.dot(xb, rk, preferred_element_type=jnp.float32)
    eidx = lax.broadcasted_iota(jnp.int32, (TB, EP), 1)
    logits = jnp.where(eidx < E, logits, -1e30)

    m = jnp.max(logits, axis=1, keepdims=True)
    ex = jnp.exp(logits - m)
    probs = ex / jnp.sum(ex, axis=1, keepdims=True)

    m1 = jnp.max(probs, axis=1, keepdims=True)
    i1 = jnp.min(jnp.where(probs == m1, eidx, EP), axis=1, keepdims=True)
    p2 = jnp.where(eidx == i1, -1.0, probs)
    m2 = jnp.max(p2, axis=1, keepdims=True)
    i2 = jnp.min(jnp.where(p2 == m2, eidx, EP), axis=1, keepdims=True)

    e_sel = jnp.where(k == 0, i1, i2)
    gate_sel = jnp.where(k == 0, m1, m2)
    mask = (eidx == e_sel).astype(jnp.float32)

    ri = lax.broadcasted_iota(jnp.int32, (TB, TB), 0)
    ci = lax.broadcasted_iota(jnp.int32, (TB, TB), 1)
    tril = (ci <= ri).astype(jnp.float32)
    inc = jnp.dot(tril, mask, preferred_element_type=jnp.float32)

    carry = carry_ref[0:1, :]
    posf = jnp.sum((inc + carry) * mask, axis=1, keepdims=True) - 1.0
    pos = posf.astype(jnp.int32)
    keep = pos < C
    slot = jnp.where(keep, e_sel * C + pos, S)
    gate = jnp.where(keep, gate_sel, 0.0)

    slot_ref[0] = jnp.broadcast_to(slot, (TB, EP))
    gate_ref[0] = jnp.broadcast_to(gate, (TB, EP))
    carry_ref[0:1, :] = carry + jnp.sum(mask, axis=0, keepdims=True)


def _route(x, rk_pad):
    return pl.pallas_call(
        _route_body,
        grid=(K * NB,),
        in_specs=[
            pl.BlockSpec((TB, D), lambda g: (g % NB, 0)),
            pl.BlockSpec((D, EP), lambda g: (0, 0)),
        ],
        out_specs=[
            pl.BlockSpec((1, TB, EP), lambda g: (g, 0, 0)),
            pl.BlockSpec((1, TB, EP), lambda g: (g, 0, 0)),
        ],
        out_shape=[
            jax.ShapeDtypeStruct((K * NB, TB, EP), jnp.int32),
            jax.ShapeDtypeStruct((K * NB, TB, EP), jnp.float32),
        ],
        scratch_shapes=[pltpu.VMEM((8, EP), jnp.float32)],
    )(x, rk_pad)


RG = 32
TG = 16
IW = 128
SR = S // IW
RPT = SR // NS


def _invert_body(slots_hbm, tokp1_hbm, st_hbm, idx_v, val_v, sh, stage_v, sem):
    cid = lax.axis_index("c")
    sid = lax.axis_index("s")

    @pl.when((cid == 0) & (sid == 0))
    def _():
        def zloop(j, _):
            stage_v[pl.ds(j * L, L)] = jnp.zeros((L,), jnp.int32)
            return 0

        lax.fori_loop(0, (S + 64) // L, zloop, 0)
        pltpu.sync_copy(stage_v, sh)

    plsc.subcore_barrier()

    @pl.when(cid == 0)
    def _():
        row0 = sid * RPT
        pltpu.sync_copy(slots_hbm.at[pl.ds(row0, RPT)], idx_v)
        pltpu.sync_copy(tokp1_hbm.at[pl.ds(row0, RPT)], val_v)

        def srow(j, _):
            pltpu.async_copy(val_v.at[j], sh.at[idx_v.at[j]], sem, add=True).wait()
            return 0

        lax.fori_loop(0, RPT, srow, 0)

    plsc.subcore_barrier()

    @pl.when(cid == 0)
    def _():
        seg = S // NS
        pltpu.sync_copy(sh.at[pl.ds(sid * seg, seg)], stage_v.at[pl.ds(0, seg)])
        pltpu.sync_copy(stage_v.at[pl.ds(0, seg)], st_hbm.at[pl.ds(sid * seg, seg)])


def _dispatch_body(x_hbm, st_hbm, out_hbm, raw_v, idx_v,
                   r0, r1, sg0, sg1, sw0, sw1):
    wid = lax.axis_index("s") * NC + lax.axis_index("c")
    per_w = S // NW
    base_w = wid * per_w
    nch = per_w // RG

    pltpu.sync_copy(st_hbm.at[pl.ds(base_w, per_w)], raw_v)

    def fix(i, _):
        v = raw_v[pl.ds(i * L, L)]
        idx_v[pl.ds(i * L, L)] = jnp.maximum(v - 1, 0)
        return 0

    lax.fori_loop(0, per_w // L, fix, 0)

    bufs = (r0, r1)
    semg = (sg0, sg1)
    semw = (sw0, sw1)
    cps_g = {}
    cps_w = {}
    cps_g[0] = pltpu.async_copy(
        x_hbm.at[idx_v.at[pl.ds(0, RG)]], bufs[0], semg[0])
    for j in range(nch):
        cps_g[j].wait()
        cps_w[j] = pltpu.async_copy(
            bufs[j % 2], out_hbm.at[pl.ds(base_w + j * RG, RG)], semw[j % 2])
        if j + 1 < nch:
            if j >= 1:
                cps_w[j - 1].wait()
            cps_g[j + 1] = pltpu.async_copy(
                x_hbm.at[idx_v.at[pl.ds((j + 1) * RG, RG)]],
                bufs[(j + 1) % 2], semg[(j + 1) % 2])
    cps_w[nch - 2].wait()
    cps_w[nch - 1].wait()


def _ffn_body(xin_ref, w1_ref, w2_ref, y_ref):
    f = pl.program_id(1)
    xb = xin_ref[0]
    w1b = w1_ref[0].astype(jnp.bfloat16)
    h = jnp.dot(xb, w1b, preferred_element_type=jnp.float32)
    hb = jnp.maximum(h, 0.0).astype(jnp.bfloat16)
    w2b = w2_ref[0].astype(jnp.bfloat16)
    acc = jnp.dot(hb, w2b, preferred_element_type=jnp.float32)

    @pl.when(f == 0)
    def _():
        y_ref[0] = acc

    @pl.when(f > 0)
    def _():
        y_ref[0] += acc


def _ffn(xin, w1, w2):
    return pl.pallas_call(
        _ffn_body,
        grid=(E, NF),
        in_specs=[
            pl.BlockSpec((1, C, D), lambda e, f: (e, 0, 0)),
            pl.BlockSpec((1, D, FB), lambda e, f: (e, 0, f)),
            pl.BlockSpec((1, FB, D), lambda e, f: (e, f, 0)),
        ],
        out_specs=pl.BlockSpec((1, C, D), lambda e, f: (e, 0, 0)),
        out_shape=jax.ShapeDtypeStruct((E, C, D), jnp.float32),
    )(xin, w1, w2)


def _combine_body(y_hbm, s0_hbm, s1_hbm, g0_hbm, g1_hbm, out_hbm,
                  i0, i1, g0, g1, a, b, o, sem0, sem1):
    wid = lax.axis_index("s") * NC + lax.axis_index("c")
    per_w = T // NW

    def chunk(j, _):
        base = wid * per_w + j * TG
        pltpu.sync_copy(s0_hbm.at[pl.ds(base, TG)], i0)
        pltpu.sync_copy(s1_hbm.at[pl.ds(base, TG)], i1)
        pltpu.sync_copy(g0_hbm.at[pl.ds(base, TG)], g0)
        pltpu.sync_copy(g1_hbm.at[pl.ds(base, TG)], g1)
        i0[...] = jnp.minimum(i0[...], S - 1)
        i1[...] = jnp.minimum(i1[...], S - 1)
        cp0 = pltpu.async_copy(y_hbm.at[i0], a, sem0)
        cp1 = pltpu.async_copy(y_hbm.at[i1], b, sem1)
        cp0.wait()
        cp1.wait()
        gv0 = g0[...]
        gv1 = g1[...]

        def row(r, _):
            rr = jnp.full((L,), r, jnp.int32)
            sg0 = gv0.at[rr].get(mode="promise_in_bounds")
            sg1 = gv1.at[rr].get(mode="promise_in_bounds")

            def col(cc, _):
                sl = pl.ds(cc * L, L)
                o[r, sl] = a[r, sl] * sg0 + b[r, sl] * sg1
                return 0

            lax.fori_loop(0, D // L, col, 0, unroll=8)
            return 0

        lax.fori_loop(0, TG, row, 0)
        pltpu.sync_copy(o, out_hbm.at[pl.ds(base, TG)])
        return 0

    lax.fori_loop(0, per_w // TG, chunk, 0)


@functools.lru_cache(maxsize=1)
def _sc_kernels():
    mesh = plsc.VectorSubcoreMesh(**_SC_MESH)
    invert = pl.kernel(
        _invert_body,
        out_type=jax.ShapeDtypeStruct((S,), jnp.int32),
        mesh=mesh,
        scratch_types=[
            pltpu.VMEM((RPT, IW), jnp.int32),
            pltpu.VMEM((RPT, IW), jnp.int32),
            pltpu.VMEM_SHARED((S + 64,), jnp.int32),
            pltpu.VMEM((S + 64,), jnp.int32),
            pltpu.SemaphoreType.DMA,
        ],
    )
    dispatch = pl.kernel(
        _dispatch_body,
        out_type=jax.ShapeDtypeStruct((S, D // 2), jnp.int32),
        mesh=mesh,
        scratch_types=[
            pltpu.VMEM((S // NW,), jnp.int32),
            pltpu.VMEM((S // NW,), jnp.int32),
            pltpu.VMEM((RG, D // 2), jnp.int32),
            pltpu.VMEM((RG, D // 2), jnp.int32),
            pltpu.SemaphoreType.DMA,
            pltpu.SemaphoreType.DMA,
            pltpu.SemaphoreType.DMA,
            pltpu.SemaphoreType.DMA,
        ],
    )
    combine = pl.kernel(
        _combine_body,
        out_type=jax.ShapeDtypeStruct((T, D), jnp.float32),
        mesh=mesh,
        scratch_types=[
            pltpu.VMEM((TG,), jnp.int32),
            pltpu.VMEM((TG,), jnp.int32),
            pltpu.VMEM((TG,), jnp.float32),
            pltpu.VMEM((TG,), jnp.float32),
            pltpu.VMEM((TG, D), jnp.float32),
            pltpu.VMEM((TG, D), jnp.float32),
            pltpu.VMEM((TG, D), jnp.float32),
            pltpu.SemaphoreType.DMA,
            pltpu.SemaphoreType.DMA,
        ],
    )
    return invert, dispatch, combine


def kernel(x, router_kernel, w1, w2):
    _invert, _dispatch, _combine = _sc_kernels()
    rk_pad = jnp.zeros((D, EP), jnp.float32).at[:, :E].set(router_kernel)
    slots3, gates3 = _route(x, rk_pad)
    slots = slots3[:, :, 0].reshape(K, T)
    gates = gates3[:, :, 0].reshape(K, T)
    tokp1 = (jnp.arange(S, dtype=jnp.int32) % T + 1).reshape(SR, IW)
    st = _invert(slots.reshape(SR, IW), tokp1)
    x_p = lax.bitcast_convert_type(
        x.astype(jnp.bfloat16).reshape(T, D // 2, 2), jnp.int32)
    xin_p = _dispatch(x_p, st)
    xin_bf = lax.bitcast_convert_type(xin_p, jnp.bfloat16).reshape(E, C, D)
    y = _ffn(xin_bf, w1, w2)
    out = _combine(y.reshape(S, D), slots[0], slots[1], gates[0], gates[1])
    return out


# speedup vs baseline: 1.4074x; 1.4074x over previous
import functools

import jax
import jax.numpy as jnp
from jax import lax
from jax.experimental import pallas as pl
from jax.experimental.pallas import tpu as pltpu
from jax.experimental.pallas import tpu_sc as plsc

E = 8
K = 2
D = 2048
F = 8192
T = 4096
C = 1024
S = E * C

TB = 512
NB = T // TB
EP = 128

FB = 512
NF = F // FB

NC, NS, L = 2, 16, 16
NW = NC * NS

_SC_MESH = dict(core_axis_name="c", subcore_axis_name="s",
                num_cores=NC, num_subcores=NS)


def _route_body(x_ref, rk_ref, slot_ref, gate_ref, xp_ref, carry_ref):
    g = pl.program_id(0)
    k = g // NB

    @pl.when(g == 0)
    def _():
        carry_ref[...] = jnp.zeros_like(carry_ref)

    xb = x_ref[...]
    rk = rk_ref[...]
    logits = jnp.dot(xb, rk, preferred_element_type=jnp.float32)
    eidx = lax.broadcasted_iota(jnp.int32, (TB, EP), 1)
    logits = jnp.where(eidx < E, logits, -1e30)

    m = jnp.max(logits, axis=1, keepdims=True)
    ex = jnp.exp(logits - m)
    probs = ex / jnp.sum(ex, axis=1, keepdims=True)

    m1 = jnp.max(probs, axis=1, keepdims=True)
    i1 = jnp.min(jnp.where(probs == m1, eidx, EP), axis=1, keepdims=True)
    p2 = jnp.where(eidx == i1, -1.0, probs)
    m2 = jnp.max(p2, axis=1, keepdims=True)
    i2 = jnp.min(jnp.where(p2 == m2, eidx, EP), axis=1, keepdims=True)

    e_sel = jnp.where(k == 0, i1, i2)
    gate_sel = jnp.where(k == 0, m1, m2)
    mask = (eidx == e_sel).astype(jnp.float32)

    ri = lax.broadcasted_iota(jnp.int32, (TB, TB), 0)
    ci = lax.broadcasted_iota(jnp.int32, (TB, TB), 1)
    tril = (ci <= ri).astype(jnp.float32)
    inc = jnp.dot(tril, mask, preferred_element_type=jnp.float32)

    carry = carry_ref[0:1, :]
    posf = jnp.sum((inc + carry) * mask, axis=1, keepdims=True) - 1.0
    pos = posf.astype(jnp.int32)
    keep = pos < C
    slot = jnp.where(keep, e_sel * C + pos, S)
    gate = jnp.where(keep, gate_sel, 0.0)

    slot_ref[0] = jnp.broadcast_to(slot, (TB, EP))
    gate_ref[0] = jnp.broadcast_to(gate, (TB, EP))
    carry_ref[0:1, :] = carry + jnp.sum(mask, axis=0, keepdims=True)

    u = lax.bitcast_convert_type(xb, jnp.int32)
    ul = u[:, :D // 2]
    ur = u[:, D // 2:]
    rl = ul + 0x7FFF + ((ul >> 16) & 1)
    rr = ur + 0x7FFF + ((ur >> 16) & 1)
    xp_ref[...] = ((rl >> 16) & 0xFFFF) | (rr & jnp.int32(-65536))


def _route(x, rk_pad):
    return pl.pallas_call(
        _route_body,
        grid=(K * NB,),
        in_specs=[
            pl.BlockSpec((TB, D), lambda g: (g % NB, 0)),
            pl.BlockSpec((D, EP), lambda g: (0, 0)),
        ],
        out_specs=[
            pl.BlockSpec((1, TB, EP), lambda g: (g, 0, 0)),
            pl.BlockSpec((1, TB, EP), lambda g: (g, 0, 0)),
            pl.BlockSpec((TB, D // 2), lambda g: (g % NB, 0)),
        ],
        out_shape=[
            jax.ShapeDtypeStruct((K * NB, TB, EP), jnp.int32),
            jax.ShapeDtypeStruct((K * NB, TB, EP), jnp.float32),
            jax.ShapeDtypeStruct((T, D // 2), jnp.int32),
        ],
        scratch_shapes=[pltpu.VMEM((8, EP), jnp.float32)],
    )(x, rk_pad)


RG = 32
TG = 16
IW = 128
SR = S // IW
RPT = SR // NS


def _invert_body(slots_hbm, tokp1_hbm, st_hbm, idx_v, val_v, sh, stage_v, sem):
    cid = lax.axis_index("c")
    sid = lax.axis_index("s")

    @pl.when((cid == 0) & (sid == 0))
    def _():
        def zloop(j, _):
            stage_v[pl.ds(j * L, L)] = jnp.zeros((L,), jnp.int32)
            return 0

        lax.fori_loop(0, (S + 64) // L, zloop, 0)
        pltpu.sync_copy(stage_v, sh)

    plsc.subcore_barrier()

    @pl.when(cid == 0)
    def _():
        row0 = sid * RPT
        pltpu.sync_copy(slots_hbm.at[pl.ds(row0, RPT)], idx_v)
        pltpu.sync_copy(tokp1_hbm.at[pl.ds(row0, RPT)], val_v)

        def srow(j, _):
            pltpu.async_copy(val_v.at[j], sh.at[idx_v.at[j]], sem, add=True).wait()
            return 0

        lax.fori_loop(0, RPT, srow, 0)

    plsc.subcore_barrier()

    @pl.when(cid == 0)
    def _():
        seg = S // NS
        pltpu.sync_copy(sh.at[pl.ds(sid * seg, seg)], stage_v.at[pl.ds(0, seg)])
        pltpu.sync_copy(stage_v.at[pl.ds(0, seg)], st_hbm.at[pl.ds(sid * seg, seg)])


def _dispatch_body(x_hbm, st_hbm, out_hbm, raw_v, idx_v,
                   r0, r1, sg0, sg1, sw0, sw1):
    wid = lax.axis_index("s") * NC + lax.axis_index("c")
    per_w = S // NW
    base_w = wid * per_w
    nch = per_w // RG

    pltpu.sync_copy(st_hbm.at[pl.ds(base_w, per_w)], raw_v)

    def fix(i, _):
        v = raw_v[pl.ds(i * L, L)]
        idx_v[pl.ds(i * L, L)] = jnp.maximum(v - 1, 0)
        return 0

    lax.fori_loop(0, per_w // L, fix, 0)

    bufs = (r0, r1)
    semg = (sg0, sg1)
    semw = (sw0, sw1)
    cps_g = {}
    cps_w = {}
    cps_g[0] = pltpu.async_copy(
        x_hbm.at[idx_v.at[pl.ds(0, RG)]], bufs[0], semg[0])
    for j in range(nch):
        cps_g[j].wait()
        cps_w[j] = pltpu.async_copy(
            bufs[j % 2], out_hbm.at[pl.ds(base_w + j * RG, RG)], semw[j % 2])
        if j + 1 < nch:
            if j >= 1:
                cps_w[j - 1].wait()
            cps_g[j + 1] = pltpu.async_copy(
                x_hbm.at[idx_v.at[pl.ds((j + 1) * RG, RG)]],
                bufs[(j + 1) % 2], semg[(j + 1) % 2])
    cps_w[nch - 2].wait()
    cps_w[nch - 1].wait()


def _ffn_body(xin_ref, w1_ref, w2_ref, y_ref):
    f = pl.program_id(1)
    p = xin_ref[0]
    xl = lax.bitcast_convert_type(p << 16, jnp.float32).astype(jnp.bfloat16)
    xr = lax.bitcast_convert_type(
        p & jnp.int32(-65536), jnp.float32).astype(jnp.bfloat16)
    w1b = w1_ref[0].astype(jnp.bfloat16)
    h = (jnp.dot(xl, w1b[:D // 2], preferred_element_type=jnp.float32)
         + jnp.dot(xr, w1b[D // 2:], preferred_element_type=jnp.float32))
    hb = jnp.maximum(h, 0.0).astype(jnp.bfloat16)
    w2b = w2_ref[0].astype(jnp.bfloat16)
    acc = jnp.dot(hb, w2b, preferred_element_type=jnp.float32)

    @pl.when(f == 0)
    def _():
        y_ref[0] = acc

    @pl.when(f > 0)
    def _():
        y_ref[0] += acc


def _ffn(xin, w1, w2):
    return pl.pallas_call(
        _ffn_body,
        grid=(E, NF),
        in_specs=[
            pl.BlockSpec((1, C, D // 2), lambda e, f: (e, 0, 0)),
            pl.BlockSpec((1, D, FB), lambda e, f: (e, 0, f)),
            pl.BlockSpec((1, FB, D), lambda e, f: (e, f, 0)),
        ],
        out_specs=pl.BlockSpec((1, C, D), lambda e, f: (e, 0, 0)),
        out_shape=jax.ShapeDtypeStruct((E, C, D), jnp.float32),
    )(xin, w1, w2)


def _combine_body(y_hbm, s0_hbm, s1_hbm, g0_hbm, g1_hbm, out_hbm,
                  i0, i1, g0, g1, a, b, o, sem0, sem1):
    wid = lax.axis_index("s") * NC + lax.axis_index("c")
    per_w = T // NW

    def chunk(j, _):
        base = wid * per_w + j * TG
        pltpu.sync_copy(s0_hbm.at[pl.ds(base, TG)], i0)
        pltpu.sync_copy(s1_hbm.at[pl.ds(base, TG)], i1)
        pltpu.sync_copy(g0_hbm.at[pl.ds(base, TG)], g0)
        pltpu.sync_copy(g1_hbm.at[pl.ds(base, TG)], g1)
        i0[...] = jnp.minimum(i0[...], S - 1)
        i1[...] = jnp.minimum(i1[...], S - 1)
        cp0 = pltpu.async_copy(y_hbm.at[i0], a, sem0)
        cp1 = pltpu.async_copy(y_hbm.at[i1], b, sem1)
        cp0.wait()
        cp1.wait()
        gv0 = g0[...]
        gv1 = g1[...]

        def row(r, _):
            rr = jnp.full((L,), r, jnp.int32)
            sg0 = gv0.at[rr].get(mode="promise_in_bounds")
            sg1 = gv1.at[rr].get(mode="promise_in_bounds")

            def col(cc, _):
                sl = pl.ds(cc * L, L)
                o[r, sl] = a[r, sl] * sg0 + b[r, sl] * sg1
                return 0

            lax.fori_loop(0, D // L, col, 0, unroll=8)
            return 0

        lax.fori_loop(0, TG, row, 0)
        pltpu.sync_copy(o, out_hbm.at[pl.ds(base, TG)])
        return 0

    lax.fori_loop(0, per_w // TG, chunk, 0)


@functools.lru_cache(maxsize=1)
def _sc_kernels():
    mesh = plsc.VectorSubcoreMesh(**_SC_MESH)
    invert = pl.kernel(
        _invert_body,
        out_type=jax.ShapeDtypeStruct((S,), jnp.int32),
        mesh=mesh,
        scratch_types=[
            pltpu.VMEM((RPT, IW), jnp.int32),
            pltpu.VMEM((RPT, IW), jnp.int32),
            pltpu.VMEM_SHARED((S + 64,), jnp.int32),
            pltpu.VMEM((S + 64,), jnp.int32),
            pltpu.SemaphoreType.DMA,
        ],
    )
    dispatch = pl.kernel(
        _dispatch_body,
        out_type=jax.ShapeDtypeStruct((S, D // 2), jnp.int32),
        mesh=mesh,
        scratch_types=[
            pltpu.VMEM((S // NW,), jnp.int32),
            pltpu.VMEM((S // NW,), jnp.int32),
            pltpu.VMEM((RG, D // 2), jnp.int32),
            pltpu.VMEM((RG, D // 2), jnp.int32),
            pltpu.SemaphoreType.DMA,
            pltpu.SemaphoreType.DMA,
            pltpu.SemaphoreType.DMA,
            pltpu.SemaphoreType.DMA,
        ],
    )
    combine = pl.kernel(
        _combine_body,
        out_type=jax.ShapeDtypeStruct((T, D), jnp.float32),
        mesh=mesh,
        scratch_types=[
            pltpu.VMEM((TG,), jnp.int32),
            pltpu.VMEM((TG,), jnp.int32),
            pltpu.VMEM((TG,), jnp.float32),
            pltpu.VMEM((TG,), jnp.float32),
            pltpu.VMEM((TG, D), jnp.float32),
            pltpu.VMEM((TG, D), jnp.float32),
            pltpu.VMEM((TG, D), jnp.float32),
            pltpu.SemaphoreType.DMA,
            pltpu.SemaphoreType.DMA,
        ],
    )
    return invert, dispatch, combine


def kernel(x, router_kernel, w1, w2):
    _invert, _dispatch, _combine = _sc_kernels()
    rk_pad = jnp.zeros((D, EP), jnp.float32).at[:, :E].set(router_kernel)
    slots3, gates3, x_p = _route(x, rk_pad)
    slots = slots3[:, :, 0].reshape(K, T)
    gates = gates3[:, :, 0].reshape(K, T)
    tokp1 = (jnp.arange(S, dtype=jnp.int32) % T + 1).reshape(SR, IW)
    st = _invert(slots.reshape(SR, IW), tokp1)
    xin_p = _dispatch(x_p, st)
    y = _ffn(xin_p.reshape(E, C, D // 2), w1, w2)
    out = _combine(y.reshape(S, D), slots[0], slots[1], gates[0], gates[1])
    return out


# speedup vs baseline: 1.4099x; 1.0018x over previous
import functools

import jax
import jax.numpy as jnp
from jax import lax
from jax.experimental import pallas as pl
from jax.experimental.pallas import tpu as pltpu
from jax.experimental.pallas import tpu_sc as plsc

E = 8
K = 2
D = 2048
F = 8192
T = 4096
C = 1024
S = E * C

TB = 512
NB = T // TB
EP = 128

FB = 512
NF = F // FB

NC, NS, L = 2, 16, 16
NW = NC * NS

_SC_MESH = dict(core_axis_name="c", subcore_axis_name="s",
                num_cores=NC, num_subcores=NS)


def _route_body(x_ref, rk_ref, slot_ref, gate_ref, xp_ref, carry_ref):
    g = pl.program_id(0)
    k = g // NB

    @pl.when(g == 0)
    def _():
        carry_ref[...] = jnp.zeros_like(carry_ref)

    xb = x_ref[...]
    rk = rk_ref[...]
    logits = jnp.dot(xb, rk, preferred_element_type=jnp.float32)
    eidx = lax.broadcasted_iota(jnp.int32, (TB, EP), 1)
    logits = jnp.where(eidx < E, logits, -1e30)

    m = jnp.max(logits, axis=1, keepdims=True)
    ex = jnp.exp(logits - m)
    probs = ex / jnp.sum(ex, axis=1, keepdims=True)

    m1 = jnp.max(probs, axis=1, keepdims=True)
    i1 = jnp.min(jnp.where(probs == m1, eidx, EP), axis=1, keepdims=True)
    p2 = jnp.where(eidx == i1, -1.0, probs)
    m2 = jnp.max(p2, axis=1, keepdims=True)
    i2 = jnp.min(jnp.where(p2 == m2, eidx, EP), axis=1, keepdims=True)

    e_sel = jnp.where(k == 0, i1, i2)
    gate_sel = jnp.where(k == 0, m1, m2)
    mask = (eidx == e_sel).astype(jnp.float32)

    ri = lax.broadcasted_iota(jnp.int32, (TB, TB), 0)
    ci = lax.broadcasted_iota(jnp.int32, (TB, TB), 1)
    tril = (ci <= ri).astype(jnp.float32)
    inc = jnp.dot(tril, mask, preferred_element_type=jnp.float32)

    carry = carry_ref[0:1, :]
    posf = jnp.sum((inc + carry) * mask, axis=1, keepdims=True) - 1.0
    pos = posf.astype(jnp.int32)
    keep = pos < C
    slot = jnp.where(keep, e_sel * C + pos, S)
    gate = jnp.where(keep, gate_sel, 0.0)

    slot_ref[0] = jnp.broadcast_to(slot, (TB, EP))
    gate_ref[0] = jnp.broadcast_to(gate, (TB, EP))
    carry_ref[0:1, :] = carry + jnp.sum(mask, axis=0, keepdims=True)

    u = lax.bitcast_convert_type(xb, jnp.int32)
    ul = u[:, :D // 2]
    ur = u[:, D // 2:]
    rl = ul + 0x7FFF + ((ul >> 16) & 1)
    rr = ur + 0x7FFF + ((ur >> 16) & 1)
    xp_ref[...] = ((rl >> 16) & 0xFFFF) | (rr & jnp.int32(-65536))


def _route(x, rk_pad):
    return pl.pallas_call(
        _route_body,
        grid=(K * NB,),
        in_specs=[
            pl.BlockSpec((TB, D), lambda g: (g % NB, 0)),
            pl.BlockSpec((D, EP), lambda g: (0, 0)),
        ],
        out_specs=[
            pl.BlockSpec((1, TB, EP), lambda g: (g, 0, 0)),
            pl.BlockSpec((1, TB, EP), lambda g: (g, 0, 0)),
            pl.BlockSpec((TB, D // 2), lambda g: (g % NB, 0)),
        ],
        out_shape=[
            jax.ShapeDtypeStruct((K * NB, TB, EP), jnp.int32),
            jax.ShapeDtypeStruct((K * NB, TB, EP), jnp.float32),
            jax.ShapeDtypeStruct((T, D // 2), jnp.int32),
        ],
        scratch_shapes=[pltpu.VMEM((8, EP), jnp.float32)],
    )(x, rk_pad)


RG = 32
TG = 16
IW = 128
SR = S // IW
RPT = SR // NS


def _invert_body(slots_hbm, tokp1_hbm, st_hbm, idx_v, val_v, sh, stage_v, sem):
    cid = lax.axis_index("c")
    sid = lax.axis_index("s")

    @pl.when((cid == 0) & (sid == 0))
    def _():
        def zloop(j, _):
            stage_v[pl.ds(j * L, L)] = jnp.zeros((L,), jnp.int32)
            return 0

        lax.fori_loop(0, (S + 64) // L, zloop, 0)
        pltpu.sync_copy(stage_v, sh)

    plsc.subcore_barrier()

    @pl.when(cid == 0)
    def _():
        row0 = sid * RPT
        pltpu.sync_copy(slots_hbm.at[pl.ds(row0, RPT)], idx_v)
        pltpu.sync_copy(tokp1_hbm.at[pl.ds(row0, RPT)], val_v)

        def srow(j, _):
            pltpu.async_copy(val_v.at[j], sh.at[idx_v.at[j]], sem, add=True).wait()
            return 0

        lax.fori_loop(0, RPT, srow, 0)

    plsc.subcore_barrier()

    @pl.when(cid == 0)
    def _():
        seg = S // NS
        pltpu.sync_copy(sh.at[pl.ds(sid * seg, seg)], stage_v.at[pl.ds(0, seg)])
        pltpu.sync_copy(stage_v.at[pl.ds(0, seg)], st_hbm.at[pl.ds(sid * seg, seg)])


def _dispatch_body(x_hbm, st_hbm, out_hbm, raw_v, idx_v,
                   r0, r1, sg0, sg1, sw0, sw1):
    wid = lax.axis_index("s") * NC + lax.axis_index("c")
    per_w = S // NW
    base_w = wid * per_w
    nch = per_w // RG

    pltpu.sync_copy(st_hbm.at[pl.ds(base_w, per_w)], raw_v)

    def fix(i, _):
        v = raw_v[pl.ds(i * L, L)]
        idx_v[pl.ds(i * L, L)] = jnp.maximum(v - 1, 0)
        return 0

    lax.fori_loop(0, per_w // L, fix, 0)

    bufs = (r0, r1)
    semg = (sg0, sg1)
    semw = (sw0, sw1)
    cps_g = {}
    cps_w = {}
    cps_g[0] = pltpu.async_copy(
        x_hbm.at[idx_v.at[pl.ds(0, RG)]], bufs[0], semg[0])
    for j in range(nch):
        cps_g[j].wait()
        cps_w[j] = pltpu.async_copy(
            bufs[j % 2], out_hbm.at[pl.ds(base_w + j * RG, RG)], semw[j % 2])
        if j + 1 < nch:
            if j >= 1:
                cps_w[j - 1].wait()
            cps_g[j + 1] = pltpu.async_copy(
                x_hbm.at[idx_v.at[pl.ds((j + 1) * RG, RG)]],
                bufs[(j + 1) % 2], semg[(j + 1) % 2])
    cps_w[nch - 2].wait()
    cps_w[nch - 1].wait()


def _ffn_body(xin_ref, w1_ref, w2_ref, y_ref, xbf_ref):
    f = pl.program_id(1)

    @pl.when(f == 0)
    def _():
        p = xin_ref[0]
        xbf_ref[:, :D // 2] = lax.bitcast_convert_type(
            p << 16, jnp.float32).astype(jnp.bfloat16)
        xbf_ref[:, D // 2:] = lax.bitcast_convert_type(
            p & jnp.int32(-65536), jnp.float32).astype(jnp.bfloat16)

    w1b = w1_ref[0].astype(jnp.bfloat16)
    h = jnp.dot(xbf_ref[...], w1b, preferred_element_type=jnp.float32)
    hb = jnp.maximum(h, 0.0).astype(jnp.bfloat16)
    w2b = w2_ref[0].astype(jnp.bfloat16)
    acc = jnp.dot(hb, w2b, preferred_element_type=jnp.float32)

    @pl.when(f == 0)
    def _():
        y_ref[0] = acc

    @pl.when(f > 0)
    def _():
        y_ref[0] += acc


def _ffn(xin, w1, w2):
    return pl.pallas_call(
        _ffn_body,
        grid=(E, NF),
        in_specs=[
            pl.BlockSpec((1, C, D // 2), lambda e, f: (e, 0, 0)),
            pl.BlockSpec((1, D, FB), lambda e, f: (e, 0, f)),
            pl.BlockSpec((1, FB, D), lambda e, f: (e, f, 0)),
        ],
        out_specs=pl.BlockSpec((1, C, D), lambda e, f: (e, 0, 0)),
        out_shape=jax.ShapeDtypeStruct((E, C, D), jnp.float32),
        scratch_shapes=[pltpu.VMEM((C, D), jnp.bfloat16)],
    )(xin, w1, w2)


def _combine_body(y_hbm, s0_hbm, s1_hbm, g0_hbm, g1_hbm, out_hbm,
                  i0, i1, g0, g1, a, b, o, sem0, sem1):
    wid = lax.axis_index("s") * NC + lax.axis_index("c")
    per_w = T // NW

    def chunk(j, _):
        base = wid * per_w + j * TG
        pltpu.sync_copy(s0_hbm.at[pl.ds(base, TG)], i0)
        pltpu.sync_copy(s1_hbm.at[pl.ds(base, TG)], i1)
        pltpu.sync_copy(g0_hbm.at[pl.ds(base, TG)], g0)
        pltpu.sync_copy(g1_hbm.at[pl.ds(base, TG)], g1)
        i0[...] = jnp.minimum(i0[...], S - 1)
        i1[...] = jnp.minimum(i1[...], S - 1)
        cp0 = pltpu.async_copy(y_hbm.at[i0], a, sem0)
        cp1 = pltpu.async_copy(y_hbm.at[i1], b, sem1)
        cp0.wait()
        cp1.wait()
        gv0 = g0[...]
        gv1 = g1[...]

        def row(r, _):
            rr = jnp.full((L,), r, jnp.int32)
            sg0 = gv0.at[rr].get(mode="promise_in_bounds")
            sg1 = gv1.at[rr].get(mode="promise_in_bounds")

            def col(cc, _):
                sl = pl.ds(cc * L, L)
                o[r, sl] = a[r, sl] * sg0 + b[r, sl] * sg1
                return 0

            lax.fori_loop(0, D // L, col, 0, unroll=8)
            return 0

        lax.fori_loop(0, TG, row, 0)
        pltpu.sync_copy(o, out_hbm.at[pl.ds(base, TG)])
        return 0

    lax.fori_loop(0, per_w // TG, chunk, 0)


@functools.lru_cache(maxsize=1)
def _sc_kernels():
    mesh = plsc.VectorSubcoreMesh(**_SC_MESH)
    invert = pl.kernel(
        _invert_body,
        out_type=jax.ShapeDtypeStruct((S,), jnp.int32),
        mesh=mesh,
        scratch_types=[
            pltpu.VMEM((RPT, IW), jnp.int32),
            pltpu.VMEM((RPT, IW), jnp.int32),
            pltpu.VMEM_SHARED((S + 64,), jnp.int32),
            pltpu.VMEM((S + 64,), jnp.int32),
            pltpu.SemaphoreType.DMA,
        ],
    )
    dispatch = pl.kernel(
        _dispatch_body,
        out_type=jax.ShapeDtypeStruct((S, D // 2), jnp.int32),
        mesh=mesh,
        scratch_types=[
            pltpu.VMEM((S // NW,), jnp.int32),
            pltpu.VMEM((S // NW,), jnp.int32),
            pltpu.VMEM((RG, D // 2), jnp.int32),
            pltpu.VMEM((RG, D // 2), jnp.int32),
            pltpu.SemaphoreType.DMA,
            pltpu.SemaphoreType.DMA,
            pltpu.SemaphoreType.DMA,
            pltpu.SemaphoreType.DMA,
        ],
    )
    combine = pl.kernel(
        _combine_body,
        out_type=jax.ShapeDtypeStruct((T, D), jnp.float32),
        mesh=mesh,
        scratch_types=[
            pltpu.VMEM((TG,), jnp.int32),
            pltpu.VMEM((TG,), jnp.int32),
            pltpu.VMEM((TG,), jnp.float32),
            pltpu.VMEM((TG,), jnp.float32),
            pltpu.VMEM((TG, D), jnp.float32),
            pltpu.VMEM((TG, D), jnp.float32),
            pltpu.VMEM((TG, D), jnp.float32),
            pltpu.SemaphoreType.DMA,
            pltpu.SemaphoreType.DMA,
        ],
    )
    return invert, dispatch, combine


def kernel(x, router_kernel, w1, w2):
    _invert, _dispatch, _combine = _sc_kernels()
    rk_pad = jnp.zeros((D, EP), jnp.float32).at[:, :E].set(router_kernel)
    slots3, gates3, x_p = _route(x, rk_pad)
    slots = slots3[:, :, 0].reshape(K, T)
    gates = gates3[:, :, 0].reshape(K, T)
    tokp1 = (jnp.arange(S, dtype=jnp.int32) % T + 1).reshape(SR, IW)
    st = _invert(slots.reshape(SR, IW), tokp1)
    xin_p = _dispatch(x_p, st)
    y = _ffn(xin_p.reshape(E, C, D // 2), w1, w2)
    out = _combine(y.reshape(S, D), slots[0], slots[1], gates[0], gates[1])
    return out


# speedup vs baseline: 1.5569x; 1.1043x over previous
import functools

import jax
import jax.numpy as jnp
from jax import lax
from jax.experimental import pallas as pl
from jax.experimental.pallas import tpu as pltpu
from jax.experimental.pallas import tpu_sc as plsc

E = 8
K = 2
D = 2048
F = 8192
T = 4096
C = 1024
S = E * C

TB = 512
NB = T // TB
EP = 128

FB = 512
NF = F // FB

NC, NS, L = 2, 16, 16
NW = NC * NS

_SC_MESH = dict(core_axis_name="c", subcore_axis_name="s",
                num_cores=NC, num_subcores=NS)


def _route_body(x_ref, rk_ref, slot_ref, gate_ref, xp_ref, carry_ref):
    g = pl.program_id(0)
    k = g // NB

    @pl.when(g == 0)
    def _():
        carry_ref[...] = jnp.zeros_like(carry_ref)

    xb = x_ref[...]
    rk = rk_ref[...]
    logits = jnp.dot(xb, rk, preferred_element_type=jnp.float32)
    eidx = lax.broadcasted_iota(jnp.int32, (TB, EP), 1)
    logits = jnp.where(eidx < E, logits, -1e30)

    m = jnp.max(logits, axis=1, keepdims=True)
    ex = jnp.exp(logits - m)
    probs = ex / jnp.sum(ex, axis=1, keepdims=True)

    m1 = jnp.max(probs, axis=1, keepdims=True)
    i1 = jnp.min(jnp.where(probs == m1, eidx, EP), axis=1, keepdims=True)
    p2 = jnp.where(eidx == i1, -1.0, probs)
    m2 = jnp.max(p2, axis=1, keepdims=True)
    i2 = jnp.min(jnp.where(p2 == m2, eidx, EP), axis=1, keepdims=True)

    e_sel = jnp.where(k == 0, i1, i2)
    gate_sel = jnp.where(k == 0, m1, m2)
    mask = (eidx == e_sel).astype(jnp.float32)

    ri = lax.broadcasted_iota(jnp.int32, (TB, TB), 0)
    ci = lax.broadcasted_iota(jnp.int32, (TB, TB), 1)
    tril = (ci <= ri).astype(jnp.float32)
    inc = jnp.dot(tril, mask, preferred_element_type=jnp.float32)

    carry = carry_ref[0:1, :]
    posf = jnp.sum((inc + carry) * mask, axis=1, keepdims=True) - 1.0
    pos = posf.astype(jnp.int32)
    keep = pos < C
    slot = jnp.where(keep, e_sel * C + pos, S)
    gate = jnp.where(keep, gate_sel, 0.0)

    slot_ref[0] = jnp.broadcast_to(slot, (TB, EP))
    gate_ref[0] = jnp.broadcast_to(gate, (TB, EP))
    carry_ref[0:1, :] = carry + jnp.sum(mask, axis=0, keepdims=True)

    u = lax.bitcast_convert_type(xb, jnp.int32)
    ul = u[:, :D // 2]
    ur = u[:, D // 2:]
    rl = ul + 0x7FFF + ((ul >> 16) & 1)
    rr = ur + 0x7FFF + ((ur >> 16) & 1)
    xp_ref[...] = ((rl >> 16) & 0xFFFF) | (rr & jnp.int32(-65536))


def _route(x, rk_pad):
    return pl.pallas_call(
        _route_body,
        grid=(K * NB,),
        in_specs=[
            pl.BlockSpec((TB, D), lambda g: (g % NB, 0)),
            pl.BlockSpec((D, EP), lambda g: (0, 0)),
        ],
        out_specs=[
            pl.BlockSpec((1, TB, EP), lambda g: (g, 0, 0)),
            pl.BlockSpec((1, TB, EP), lambda g: (g, 0, 0)),
            pl.BlockSpec((TB, D // 2), lambda g: (g % NB, 0)),
        ],
        out_shape=[
            jax.ShapeDtypeStruct((K * NB, TB, EP), jnp.int32),
            jax.ShapeDtypeStruct((K * NB, TB, EP), jnp.float32),
            jax.ShapeDtypeStruct((T, D // 2), jnp.int32),
        ],
        scratch_shapes=[pltpu.VMEM((8, EP), jnp.float32)],
    )(x, rk_pad)


RG = 32
TG = 16
IW = 128
SR = S // IW
RPT = SR // NS


def _invert_body(slots_hbm, tokp1_hbm, st_hbm, idx_v, val_v, sh, stage_v, sem):
    cid = lax.axis_index("c")
    sid = lax.axis_index("s")

    @pl.when((cid == 0) & (sid == 0))
    def _():
        def zloop(j, _):
            stage_v[pl.ds(j * L, L)] = jnp.zeros((L,), jnp.int32)
            return 0

        lax.fori_loop(0, (S + 64) // L, zloop, 0)
        pltpu.sync_copy(stage_v, sh)

    plsc.subcore_barrier()

    @pl.when(cid == 0)
    def _():
        row0 = sid * RPT
        pltpu.sync_copy(slots_hbm.at[pl.ds(row0, RPT)], idx_v)
        pltpu.sync_copy(tokp1_hbm.at[pl.ds(row0, RPT)], val_v)

        def srow(j, _):
            pltpu.async_copy(val_v.at[j], sh.at[idx_v.at[j]], sem, add=True).wait()
            return 0

        lax.fori_loop(0, RPT, srow, 0)

    plsc.subcore_barrier()

    @pl.when(cid == 0)
    def _():
        seg = S // NS
        pltpu.sync_copy(sh.at[pl.ds(sid * seg, seg)], stage_v.at[pl.ds(0, seg)])
        pltpu.sync_copy(stage_v.at[pl.ds(0, seg)], st_hbm.at[pl.ds(sid * seg, seg)])


def _dispatch_body(x_hbm, st_hbm, out_hbm, raw_v, idx_v,
                   r0, r1, sg0, sg1, sw0, sw1):
    wid = lax.axis_index("s") * NC + lax.axis_index("c")
    per_w = S // NW
    base_w = wid * per_w
    nch = per_w // RG

    pltpu.sync_copy(st_hbm.at[pl.ds(base_w, per_w)], raw_v)

    def fix(i, _):
        v = raw_v[pl.ds(i * L, L)]
        idx_v[pl.ds(i * L, L)] = jnp.maximum(v - 1, 0)
        return 0

    lax.fori_loop(0, per_w // L, fix, 0)

    bufs = (r0, r1)
    semg = (sg0, sg1)
    semw = (sw0, sw1)
    cps_g = {}
    cps_w = {}
    cps_g[0] = pltpu.async_copy(
        x_hbm.at[idx_v.at[pl.ds(0, RG)]], bufs[0], semg[0])
    for j in range(nch):
        cps_g[j].wait()
        cps_w[j] = pltpu.async_copy(
            bufs[j % 2], out_hbm.at[pl.ds(base_w + j * RG, RG)], semw[j % 2])
        if j + 1 < nch:
            if j >= 1:
                cps_w[j - 1].wait()
            cps_g[j + 1] = pltpu.async_copy(
                x_hbm.at[idx_v.at[pl.ds((j + 1) * RG, RG)]],
                bufs[(j + 1) % 2], semg[(j + 1) % 2])
    cps_w[nch - 2].wait()
    cps_w[nch - 1].wait()


def _ffn_body(xin_ref, w1_ref, w2_ref, y_ref, xbf_ref, acc_ref):
    f = pl.program_id(1)

    @pl.when(f == 0)
    def _():
        p = xin_ref[0]
        xbf_ref[:, :D // 2] = lax.bitcast_convert_type(
            p << 16, jnp.float32).astype(jnp.bfloat16)
        xbf_ref[:, D // 2:] = lax.bitcast_convert_type(
            p & jnp.int32(-65536), jnp.float32).astype(jnp.bfloat16)

    w1b = w1_ref[0].astype(jnp.bfloat16)
    h = jnp.dot(xbf_ref[...], w1b, preferred_element_type=jnp.float32)
    hb = jnp.maximum(h, 0.0).astype(jnp.bfloat16)
    w2b = w2_ref[0].astype(jnp.bfloat16)
    acc = jnp.dot(hb, w2b, preferred_element_type=jnp.float32)

    @pl.when(f == 0)
    def _():
        acc_ref[...] = acc

    @pl.when(f > 0)
    def _():
        acc_ref[...] += acc

    @pl.when(f == NF - 1)
    def _():
        v = lax.bitcast_convert_type(acc_ref[...], jnp.int32)
        vl = v[:, :D // 2]
        vr = v[:, D // 2:]
        ql = vl + 0x7FFF + ((vl >> 16) & 1)
        qr = vr + 0x7FFF + ((vr >> 16) & 1)
        y_ref[0] = ((ql >> 16) & 0xFFFF) | (qr & jnp.int32(-65536))


def _ffn(xin, w1, w2):
    return pl.pallas_call(
        _ffn_body,
        grid=(E, NF),
        in_specs=[
            pl.BlockSpec((1, C, D // 2), lambda e, f: (e, 0, 0)),
            pl.BlockSpec((1, D, FB), lambda e, f: (e, 0, f)),
            pl.BlockSpec((1, FB, D), lambda e, f: (e, f, 0)),
        ],
        out_specs=pl.BlockSpec((1, C, D // 2), lambda e, f: (e, 0, 0)),
        out_shape=jax.ShapeDtypeStruct((E, C, D // 2), jnp.int32),
        scratch_shapes=[
            pltpu.VMEM((C, D), jnp.bfloat16),
            pltpu.VMEM((C, D), jnp.float32),
        ],
    )(xin, w1, w2)


def _cgather_body(y_hbm, s0_hbm, s1_hbm, ga_hbm, gb_hbm,
                  i0, i1, a0, a1, b0, b1,
                  sga0, sga1, sgb0, sgb1, swa0, swa1, swb0, swb1):
    wid = lax.axis_index("s") * NC + lax.axis_index("c")
    per_w = T // NW
    base_w = wid * per_w
    nch = per_w // TG

    pltpu.sync_copy(s0_hbm.at[pl.ds(base_w, per_w)], i0)
    pltpu.sync_copy(s1_hbm.at[pl.ds(base_w, per_w)], i1)

    def fix(i, _):
        sl = pl.ds(i * L, L)
        i0[sl] = jnp.minimum(i0[sl], S - 1)
        i1[sl] = jnp.minimum(i1[sl], S - 1)
        return 0

    lax.fori_loop(0, per_w // L, fix, 0)

    abufs = (a0, a1)
    bbufs = (b0, b1)
    sgas = (sga0, sga1)
    sgbs = (sgb0, sgb1)
    swas = (swa0, swa1)
    swbs = (swb0, swb1)

    def ga_start(j):
        return pltpu.async_copy(
            y_hbm.at[i0.at[pl.ds(j * TG, TG)]], abufs[j % 2], sgas[j % 2])

    def gb_start(j):
        return pltpu.async_copy(
            y_hbm.at[i1.at[pl.ds(j * TG, TG)]], bbufs[j % 2], sgbs[j % 2])

    cga = {0: ga_start(0)}
    cgb = {0: gb_start(0)}
    cwa = {}
    cwb = {}
    for j in range(nch):
        sl = j % 2
        cga[j].wait()
        cwa[j] = pltpu.async_copy(
            abufs[sl], ga_hbm.at[pl.ds(base_w + j * TG, TG)], swas[sl])
        cgb[j].wait()
        cwb[j] = pltpu.async_copy(
            bbufs[sl], gb_hbm.at[pl.ds(base_w + j * TG, TG)], swbs[sl])
        if j + 1 < nch:
            if j >= 1:
                cwa[j - 1].wait()
                cwb[j - 1].wait()
            cga[j + 1] = ga_start(j + 1)
            cgb[j + 1] = gb_start(j + 1)
    cwa[nch - 2].wait()
    cwa[nch - 1].wait()
    cwb[nch - 2].wait()
    cwb[nch - 1].wait()


def _cmix_body(ga_ref, gb_ref, g0_ref, g1_ref, out_ref):
    m_hi = jnp.int32(-65536)
    va = ga_ref[...]
    vb = gb_ref[...]
    alo = lax.bitcast_convert_type(va << 16, jnp.float32)
    ahi = lax.bitcast_convert_type(va & m_hi, jnp.float32)
    blo = lax.bitcast_convert_type(vb << 16, jnp.float32)
    bhi = lax.bitcast_convert_type(vb & m_hi, jnp.float32)
    s0 = g0_ref[0, :, 0:1]
    s1 = g1_ref[0, :, 0:1]
    out_ref[:, :D // 2] = alo * s0 + blo * s1
    out_ref[:, D // 2:] = ahi * s0 + bhi * s1


def _cmix(ga, gb, gates3):
    return pl.pallas_call(
        _cmix_body,
        grid=(NB,),
        in_specs=[
            pl.BlockSpec((TB, D // 2), lambda g: (g, 0)),
            pl.BlockSpec((TB, D // 2), lambda g: (g, 0)),
            pl.BlockSpec((1, TB, EP), lambda g: (g, 0, 0)),
            pl.BlockSpec((1, TB, EP), lambda g: (g + NB, 0, 0)),
        ],
        out_specs=pl.BlockSpec((TB, D), lambda g: (g, 0)),
        out_shape=jax.ShapeDtypeStruct((T, D), jnp.float32),
    )(ga, gb, gates3, gates3)


@functools.lru_cache(maxsize=1)
def _sc_kernels():
    mesh = plsc.VectorSubcoreMesh(**_SC_MESH)
    invert = pl.kernel(
        _invert_body,
        out_type=jax.ShapeDtypeStruct((S,), jnp.int32),
        mesh=mesh,
        scratch_types=[
            pltpu.VMEM((RPT, IW), jnp.int32),
            pltpu.VMEM((RPT, IW), jnp.int32),
            pltpu.VMEM_SHARED((S + 64,), jnp.int32),
            pltpu.VMEM((S + 64,), jnp.int32),
            pltpu.SemaphoreType.DMA,
        ],
    )
    dispatch = pl.kernel(
        _dispatch_body,
        out_type=jax.ShapeDtypeStruct((S, D // 2), jnp.int32),
        mesh=mesh,
        scratch_types=[
            pltpu.VMEM((S // NW,), jnp.int32),
            pltpu.VMEM((S // NW,), jnp.int32),
            pltpu.VMEM((RG, D // 2), jnp.int32),
            pltpu.VMEM((RG, D // 2), jnp.int32),
            pltpu.SemaphoreType.DMA,
            pltpu.SemaphoreType.DMA,
            pltpu.SemaphoreType.DMA,
            pltpu.SemaphoreType.DMA,
        ],
    )
    cgather = pl.kernel(
        _cgather_body,
        out_type=(
            jax.ShapeDtypeStruct((T, D // 2), jnp.int32),
            jax.ShapeDtypeStruct((T, D // 2), jnp.int32),
        ),
        mesh=mesh,
        scratch_types=[
            pltpu.VMEM((T // NW,), jnp.int32),
            pltpu.VMEM((T // NW,), jnp.int32),
            pltpu.VMEM((TG, D // 2), jnp.int32),
            pltpu.VMEM((TG, D // 2), jnp.int32),
            pltpu.VMEM((TG, D // 2), jnp.int32),
            pltpu.VMEM((TG, D // 2), jnp.int32),
            pltpu.SemaphoreType.DMA,
            pltpu.SemaphoreType.DMA,
            pltpu.SemaphoreType.DMA,
            pltpu.SemaphoreType.DMA,
            pltpu.SemaphoreType.DMA,
            pltpu.SemaphoreType.DMA,
            pltpu.SemaphoreType.DMA,
            pltpu.SemaphoreType.DMA,
        ],
    )
    return invert, dispatch, cgather


def kernel(x, router_kernel, w1, w2):
    _invert, _dispatch, _cgather = _sc_kernels()
    rk_pad = jnp.zeros((D, EP), jnp.float32).at[:, :E].set(router_kernel)
    slots3, gates3, x_p = _route(x, rk_pad)
    slots = slots3[:, :, 0].reshape(K, T)
    tokp1 = (jnp.arange(S, dtype=jnp.int32) % T + 1).reshape(SR, IW)
    st = _invert(slots.reshape(SR, IW), tokp1)
    xin_p = _dispatch(x_p, st)
    y = _ffn(xin_p.reshape(E, C, D // 2), w1, w2)
    ga, gb = _cgather(y.reshape(S, D // 2), slots[0], slots[1])
    return _cmix(ga, gb, gates3)


# speedup vs baseline: 1.5654x; 1.0055x over previous
import functools

import jax
import jax.numpy as jnp
from jax import lax
from jax.experimental import pallas as pl
from jax.experimental.pallas import tpu as pltpu
from jax.experimental.pallas import tpu_sc as plsc

E = 8
K = 2
D = 2048
F = 8192
T = 4096
C = 1024
S = E * C

TB = 512
NB = T // TB
EP = 128

FB = 512
NF = F // FB

NC, NS, L = 2, 16, 16
NW = NC * NS

_SC_MESH = dict(core_axis_name="c", subcore_axis_name="s",
                num_cores=NC, num_subcores=NS)


def _topk_body(x_ref, rk_ref, tp_ref, gp_ref, xp_ref):
    xb = x_ref[...]
    rk = rk_ref[...]
    logits = jnp.dot(xb, rk, preferred_element_type=jnp.float32)
    eidx = lax.broadcasted_iota(jnp.int32, (TB, EP), 1)
    logits = jnp.where(eidx < E, logits, -1e30)

    m = jnp.max(logits, axis=1, keepdims=True)
    ex = jnp.exp(logits - m)
    probs = ex / jnp.sum(ex, axis=1, keepdims=True)

    m1 = jnp.max(probs, axis=1, keepdims=True)
    i1 = jnp.min(jnp.where(probs == m1, eidx, EP), axis=1, keepdims=True)
    p2 = jnp.where(eidx == i1, -1.0, probs)
    m2 = jnp.max(p2, axis=1, keepdims=True)
    i2 = jnp.min(jnp.where(p2 == m2, eidx, EP), axis=1, keepdims=True)

    tp_ref[0] = jnp.broadcast_to(i1 | (i2 << 8), (TB, EP))

    def _bf16_bits(v):
        u = lax.bitcast_convert_type(v, jnp.int32)
        return ((u + 0x7FFF + ((u >> 16) & 1)) >> 16) & 0xFFFF

    gpk = (_bf16_bits(m1) << 16) | _bf16_bits(m2)
    gp_ref[0] = jnp.broadcast_to(gpk, (TB, EP))

    u = lax.bitcast_convert_type(xb, jnp.int32)
    ul = u[:, :D // 2]
    ur = u[:, D // 2:]
    rl = ul + 0x7FFF + ((ul >> 16) & 1)
    rr = ur + 0x7FFF + ((ur >> 16) & 1)
    xp_ref[...] = ((rl >> 16) & 0xFFFF) | (rr & jnp.int32(-65536))


def _topk(x, rk_pad):
    return pl.pallas_call(
        _topk_body,
        grid=(NB,),
        in_specs=[
            pl.BlockSpec((TB, D), lambda g: (g, 0)),
            pl.BlockSpec((D, EP), lambda g: (0, 0)),
        ],
        out_specs=[
            pl.BlockSpec((1, TB, EP), lambda g: (g, 0, 0)),
            pl.BlockSpec((1, TB, EP), lambda g: (g, 0, 0)),
            pl.BlockSpec((TB, D // 2), lambda g: (g, 0)),
        ],
        out_shape=[
            jax.ShapeDtypeStruct((NB, TB, EP), jnp.int32),
            jax.ShapeDtypeStruct((NB, TB, EP), jnp.int32),
            jax.ShapeDtypeStruct((T, D // 2), jnp.int32),
        ],
    )(x, rk_pad)


def _cumsum_body(tp_ref, gp_ref, slot_ref, gate_ref, carry_ref):
    g = pl.program_id(0)
    k = g // NB

    @pl.when(g == 0)
    def _():
        carry_ref[...] = jnp.zeros_like(carry_ref)

    eidx = lax.broadcasted_iota(jnp.int32, (TB, EP), 1)
    tpk = tp_ref[0, :, 0:1]
    gpk = gp_ref[0, :, 0:1]
    e_sel = jnp.where(k == 0, tpk & 0xFF, tpk >> 8)
    gate_sel = lax.bitcast_convert_type(
        jnp.where(k == 0, gpk & jnp.int32(-65536), gpk << 16), jnp.float32)
    mask = (eidx == e_sel).astype(jnp.float32)

    ri = lax.broadcasted_iota(jnp.int32, (TB, TB), 0)
    ci = lax.broadcasted_iota(jnp.int32, (TB, TB), 1)
    tril = (ci <= ri).astype(jnp.float32)
    inc = jnp.dot(tril, mask, preferred_element_type=jnp.float32)

    carry = carry_ref[0:1, :]
    posf = jnp.sum((inc + carry) * mask, axis=1, keepdims=True) - 1.0
    pos = posf.astype(jnp.int32)
    keep = pos < C
    slot = jnp.where(keep, e_sel * C + pos, S)
    gate = jnp.where(keep, gate_sel, 0.0)

    slot_ref[0] = jnp.broadcast_to(slot, (TB, EP))
    gate_ref[0] = jnp.broadcast_to(gate, (TB, EP))
    carry_ref[0:1, :] = carry + jnp.sum(mask, axis=0, keepdims=True)


def _cumsum(tp, gp):
    return pl.pallas_call(
        _cumsum_body,
        grid=(K * NB,),
        in_specs=[
            pl.BlockSpec((1, TB, EP), lambda g: (g % NB, 0, 0)),
            pl.BlockSpec((1, TB, EP), lambda g: (g % NB, 0, 0)),
        ],
        out_specs=[
            pl.BlockSpec((1, TB, EP), lambda g: (g, 0, 0)),
            pl.BlockSpec((1, TB, EP), lambda g: (g, 0, 0)),
        ],
        out_shape=[
            jax.ShapeDtypeStruct((K * NB, TB, EP), jnp.int32),
            jax.ShapeDtypeStruct((K * NB, TB, EP), jnp.float32),
        ],
        scratch_shapes=[pltpu.VMEM((8, EP), jnp.float32)],
    )(tp, gp)


RG = 32
TG = 16
IW = 128
SR = S // IW
RPT = SR // NS


def _invert_body(slots_hbm, tokp1_hbm, st_hbm, idx_v, val_v, sh, stage_v, sem):
    cid = lax.axis_index("c")
    sid = lax.axis_index("s")

    @pl.when((cid == 0) & (sid == 0))
    def _():
        def zloop(j, _):
            stage_v[pl.ds(j * L, L)] = jnp.zeros((L,), jnp.int32)
            return 0

        lax.fori_loop(0, (S + 64) // L, zloop, 0)
        pltpu.sync_copy(stage_v, sh)

    plsc.subcore_barrier()

    @pl.when(cid == 0)
    def _():
        row0 = sid * RPT
        pltpu.sync_copy(slots_hbm.at[pl.ds(row0, RPT)], idx_v)
        pltpu.sync_copy(tokp1_hbm.at[pl.ds(row0, RPT)], val_v)

        def srow(j, _):
            pltpu.async_copy(val_v.at[j], sh.at[idx_v.at[j]], sem, add=True).wait()
            return 0

        lax.fori_loop(0, RPT, srow, 0)

    plsc.subcore_barrier()

    @pl.when(cid == 0)
    def _():
        seg = S // NS
        pltpu.sync_copy(sh.at[pl.ds(sid * seg, seg)], stage_v.at[pl.ds(0, seg)])
        pltpu.sync_copy(stage_v.at[pl.ds(0, seg)], st_hbm.at[pl.ds(sid * seg, seg)])


def _dispatch_body(x_hbm, st_hbm, out_hbm, raw_v, idx_v,
                   r0, r1, sg0, sg1, sw0, sw1):
    wid = lax.axis_index("s") * NC + lax.axis_index("c")
    per_w = S // NW
    base_w = wid * per_w
    nch = per_w // RG

    pltpu.sync_copy(st_hbm.at[pl.ds(base_w, per_w)], raw_v)

    def fix(i, _):
        v = raw_v[pl.ds(i * L, L)]
        idx_v[pl.ds(i * L, L)] = jnp.maximum(v - 1, 0)
        return 0

    lax.fori_loop(0, per_w // L, fix, 0)

    bufs = (r0, r1)
    semg = (sg0, sg1)
    semw = (sw0, sw1)
    cps_g = {}
    cps_w = {}
    cps_g[0] = pltpu.async_copy(
        x_hbm.at[idx_v.at[pl.ds(0, RG)]], bufs[0], semg[0])
    for j in range(nch):
        cps_g[j].wait()
        cps_w[j] = pltpu.async_copy(
            bufs[j % 2], out_hbm.at[pl.ds(base_w + j * RG, RG)], semw[j % 2])
        if j + 1 < nch:
            if j >= 1:
                cps_w[j - 1].wait()
            cps_g[j + 1] = pltpu.async_copy(
                x_hbm.at[idx_v.at[pl.ds((j + 1) * RG, RG)]],
                bufs[(j + 1) % 2], semg[(j + 1) % 2])
    cps_w[nch - 2].wait()
    cps_w[nch - 1].wait()


def _ffn_body(xin_ref, w1_ref, w2_ref, y_ref, xbf_ref, acc_ref):
    f = pl.program_id(1)

    @pl.when(f == 0)
    def _():
        p = xin_ref[0]
        xbf_ref[:, :D // 2] = lax.bitcast_convert_type(
            p << 16, jnp.float32).astype(jnp.bfloat16)
        xbf_ref[:, D // 2:] = lax.bitcast_convert_type(
            p & jnp.int32(-65536), jnp.float32).astype(jnp.bfloat16)

    w1b = w1_ref[0].astype(jnp.bfloat16)
    h = jnp.dot(xbf_ref[...], w1b, preferred_element_type=jnp.float32)
    hb = jnp.maximum(h, 0.0).astype(jnp.bfloat16)
    w2b = w2_ref[0].astype(jnp.bfloat16)
    acc = jnp.dot(hb, w2b, preferred_element_type=jnp.float32)

    @pl.when(f == 0)
    def _():
        acc_ref[...] = acc

    @pl.when(f > 0)
    def _():
        acc_ref[...] += acc

    @pl.when(f == NF - 1)
    def _():
        v = lax.bitcast_convert_type(acc_ref[...], jnp.int32)
        vl = v[:, :D // 2]
        vr = v[:, D // 2:]
        ql = vl + 0x7FFF + ((vl >> 16) & 1)
        qr = vr + 0x7FFF + ((vr >> 16) & 1)
        y_ref[0] = ((ql >> 16) & 0xFFFF) | (qr & jnp.int32(-65536))


def _ffn(xin, w1, w2):
    return pl.pallas_call(
        _ffn_body,
        grid=(E, NF),
        in_specs=[
            pl.BlockSpec((1, C, D // 2), lambda e, f: (e, 0, 0)),
            pl.BlockSpec((1, D, FB), lambda e, f: (e, 0, f)),
            pl.BlockSpec((1, FB, D), lambda e, f: (e, f, 0)),
        ],
        out_specs=pl.BlockSpec((1, C, D // 2), lambda e, f: (e, 0, 0)),
        out_shape=jax.ShapeDtypeStruct((E, C, D // 2), jnp.int32),
        scratch_shapes=[
            pltpu.VMEM((C, D), jnp.bfloat16),
            pltpu.VMEM((C, D), jnp.float32),
        ],
    )(xin, w1, w2)


def _cgather_body(y_hbm, s0_hbm, s1_hbm, ga_hbm, gb_hbm,
                  i0, i1, a0, a1, b0, b1,
                  sga0, sga1, sgb0, sgb1, swa0, swa1, swb0, swb1):
    wid = lax.axis_index("s") * NC + lax.axis_index("c")
    per_w = T // NW
    base_w = wid * per_w
    nch = per_w // TG

    pltpu.sync_copy(s0_hbm.at[pl.ds(base_w, per_w)], i0)
    pltpu.sync_copy(s1_hbm.at[pl.ds(base_w, per_w)], i1)

    def fix(i, _):
        sl = pl.ds(i * L, L)
        i0[sl] = jnp.minimum(i0[sl], S - 1)
        i1[sl] = jnp.minimum(i1[sl], S - 1)
        return 0

    lax.fori_loop(0, per_w // L, fix, 0)

    abufs = (a0, a1)
    bbufs = (b0, b1)
    sgas = (sga0, sga1)
    sgbs = (sgb0, sgb1)
    swas = (swa0, swa1)
    swbs = (swb0, swb1)

    def ga_start(j):
        return pltpu.async_copy(
            y_hbm.at[i0.at[pl.ds(j * TG, TG)]], abufs[j % 2], sgas[j % 2])

    def gb_start(j):
        return pltpu.async_copy(
            y_hbm.at[i1.at[pl.ds(j * TG, TG)]], bbufs[j % 2], sgbs[j % 2])

    cga = {0: ga_start(0)}
    cgb = {0: gb_start(0)}
    cwa = {}
    cwb = {}
    for j in range(nch):
        sl = j % 2
        cga[j].wait()
        cwa[j] = pltpu.async_copy(
            abufs[sl], ga_hbm.at[pl.ds(base_w + j * TG, TG)], swas[sl])
        cgb[j].wait()
        cwb[j] = pltpu.async_copy(
            bbufs[sl], gb_hbm.at[pl.ds(base_w + j * TG, TG)], swbs[sl])
        if j + 1 < nch:
            if j >= 1:
                cwa[j - 1].wait()
                cwb[j - 1].wait()
            cga[j + 1] = ga_start(j + 1)
            cgb[j + 1] = gb_start(j + 1)
    cwa[nch - 2].wait()
    cwa[nch - 1].wait()
    cwb[nch - 2].wait()
    cwb[nch - 1].wait()


def _cmix_body(ga_ref, gb_ref, g0_ref, g1_ref, out_ref):
    m_hi = jnp.int32(-65536)
    va = ga_ref[...]
    vb = gb_ref[...]
    alo = lax.bitcast_convert_type(va << 16, jnp.float32)
    ahi = lax.bitcast_convert_type(va & m_hi, jnp.float32)
    blo = lax.bitcast_convert_type(vb << 16, jnp.float32)
    bhi = lax.bitcast_convert_type(vb & m_hi, jnp.float32)
    s0 = g0_ref[0, :, 0:1]
    s1 = g1_ref[0, :, 0:1]
    out_ref[:, :D // 2] = alo * s0 + blo * s1
    out_ref[:, D // 2:] = ahi * s0 + bhi * s1


def _cmix(ga, gb, gates3):
    return pl.pallas_call(
        _cmix_body,
        grid=(NB,),
        in_specs=[
            pl.BlockSpec((TB, D // 2), lambda g: (g, 0)),
            pl.BlockSpec((TB, D // 2), lambda g: (g, 0)),
            pl.BlockSpec((1, TB, EP), lambda g: (g, 0, 0)),
            pl.BlockSpec((1, TB, EP), lambda g: (g + NB, 0, 0)),
        ],
        out_specs=pl.BlockSpec((TB, D), lambda g: (g, 0)),
        out_shape=jax.ShapeDtypeStruct((T, D), jnp.float32),
    )(ga, gb, gates3, gates3)


@functools.lru_cache(maxsize=1)
def _sc_kernels():
    mesh = plsc.VectorSubcoreMesh(**_SC_MESH)
    invert = pl.kernel(
        _invert_body,
        out_type=jax.ShapeDtypeStruct((S,), jnp.int32),
        mesh=mesh,
        scratch_types=[
            pltpu.VMEM((RPT, IW), jnp.int32),
            pltpu.VMEM((RPT, IW), jnp.int32),
            pltpu.VMEM_SHARED((S + 64,), jnp.int32),
            pltpu.VMEM((S + 64,), jnp.int32),
            pltpu.SemaphoreType.DMA,
        ],
    )
    dispatch = pl.kernel(
        _dispatch_body,
        out_type=jax.ShapeDtypeStruct((S, D // 2), jnp.int32),
        mesh=mesh,
        scratch_types=[
            pltpu.VMEM((S // NW,), jnp.int32),
            pltpu.VMEM((S // NW,), jnp.int32),
            pltpu.VMEM((RG, D // 2), jnp.int32),
            pltpu.VMEM((RG, D // 2), jnp.int32),
            pltpu.SemaphoreType.DMA,
            pltpu.SemaphoreType.DMA,
            pltpu.SemaphoreType.DMA,
            pltpu.SemaphoreType.DMA,
        ],
    )
    cgather = pl.kernel(
        _cgather_body,
        out_type=(
            jax.ShapeDtypeStruct((T, D // 2), jnp.int32),
            jax.ShapeDtypeStruct((T, D // 2), jnp.int32),
        ),
        mesh=mesh,
        scratch_types=[
            pltpu.VMEM((T // NW,), jnp.int32),
            pltpu.VMEM((T // NW,), jnp.int32),
            pltpu.VMEM((TG, D // 2), jnp.int32),
            pltpu.VMEM((TG, D // 2), jnp.int32),
            pltpu.VMEM((TG, D // 2), jnp.int32),
            pltpu.VMEM((TG, D // 2), jnp.int32),
            pltpu.SemaphoreType.DMA,
            pltpu.SemaphoreType.DMA,
            pltpu.SemaphoreType.DMA,
            pltpu.SemaphoreType.DMA,
            pltpu.SemaphoreType.DMA,
            pltpu.SemaphoreType.DMA,
            pltpu.SemaphoreType.DMA,
            pltpu.SemaphoreType.DMA,
        ],
    )
    return invert, dispatch, cgather


def kernel(x, router_kernel, w1, w2):
    _invert, _dispatch, _cgather = _sc_kernels()
    rk_pad = jnp.zeros((D, EP), jnp.float32).at[:, :E].set(router_kernel)
    tp, gp, x_p = _topk(x, rk_pad)
    slots3, gates3 = _cumsum(tp, gp)
    slots = slots3[:, :, 0].reshape(K, T)
    tokp1 = (jnp.arange(S, dtype=jnp.int32) % T + 1).reshape(SR, IW)
    st = _invert(slots.reshape(SR, IW), tokp1)
    xin_p = _dispatch(x_p, st)
    y = _ffn(xin_p.reshape(E, C, D // 2), w1, w2)
    ga, gb = _cgather(y.reshape(S, D // 2), slots[0], slots[1])
    return _cmix(ga, gb, gates3)
